# baseline (speedup 1.0000x reference)
import functools

import jax
import jax.numpy as jnp
from jax.experimental import pallas as pl
from jax.experimental.pallas import tpu as pltpu

_N_BLK = 2048
_SCALE = 2.0


def _fused_body(idx_ref, x_ref, w_ref, a_ref, bf_ref, o_ref, wb_ref, xu_ref,
                *, d_in):
    @pl.when(pl.program_id(0) == 0)
    def _fill_wb():
        wb_ref[:, :d_in] = w_ref[...].astype(jnp.bfloat16)
        wb_ref[:, d_in:] = bf_ref[...].astype(jnp.bfloat16)

    x = x_ref[...].astype(jnp.bfloat16)
    xu_ref[:, :d_in] = x
    u = jax.lax.dot_general(
        x, a_ref[...], (((1,), (1,)), ((), ())),
        preferred_element_type=jnp.float32)
    idx = jnp.reshape(idx_ref[0, 0, :], (idx_ref.shape[2], 1))
    col = jax.lax.broadcasted_iota(jnp.int32, u.shape, 1)
    xu_ref[:, d_in:] = jnp.where(
        (col >> 4) == idx, u, 0.0).astype(jnp.bfloat16)
    o_ref[...] = jax.lax.dot_general(
        xu_ref[...], wb_ref[...], (((1,), (1,)), ((), ())),
        preferred_element_type=jnp.float32)


def kernel(x, active_lora_indices, W, lora_A_pool, lora_B_pool):
    n, d_in = x.shape
    d_out = W.shape[0]
    L, r, _ = lora_A_pool.shape
    lr = L * r
    nb = n // _N_BLK
    a_cat = lora_A_pool.reshape(lr, d_in).astype(jnp.bfloat16)
    b_flat = (_SCALE * jnp.transpose(lora_B_pool, (1, 0, 2))).reshape(d_out, lr)
    idx3 = active_lora_indices.reshape(nb, 1, _N_BLK)
    return pl.pallas_call(
        functools.partial(_fused_body, d_in=d_in),
        grid=(nb,),
        in_specs=[
            pl.BlockSpec((1, 1, _N_BLK), lambda i: (i, 0, 0)),
            pl.BlockSpec((_N_BLK, d_in), lambda i: (i, 0)),
            pl.BlockSpec((d_out, d_in), lambda i: (0, 0)),
            pl.BlockSpec((lr, d_in), lambda i: (0, 0)),
            pl.BlockSpec((d_out, lr), lambda i: (0, 0)),
        ],
        out_specs=pl.BlockSpec((_N_BLK, d_out), lambda i: (i, 0)),
        out_shape=jax.ShapeDtypeStruct((n, d_out), jnp.float32),
        scratch_shapes=[pltpu.VMEM((d_out, d_in + lr), jnp.bfloat16),
                        pltpu.VMEM((_N_BLK, d_in + lr), jnp.bfloat16)],
        compiler_params=pltpu.CompilerParams(
            dimension_semantics=("arbitrary",)),
    )(idx3, x, W, a_cat, b_flat)


# speedup vs baseline: 1.0253x; 1.0253x over previous
import functools

import jax
import jax.numpy as jnp
from jax.experimental import pallas as pl
from jax.experimental.pallas import tpu as pltpu

_N_BLK = 2048
_SCALE = 2.0


def _fused_body(idx_ref, x_ref, w_ref, a_ref, bf_ref, o_ref, wb_ref, xu_ref,
                *, d_in):
    @pl.when(pl.program_id(0) == 0)
    def _fill_wb():
        wb_ref[:, :d_in] = w_ref[...]
        wb_ref[:, d_in:] = bf_ref[...]

    x = x_ref[...]
    xu_ref[:, :d_in] = x
    u = jax.lax.dot_general(
        x, a_ref[...], (((1,), (1,)), ((), ())),
        preferred_element_type=jnp.float32)
    idx = jnp.reshape(idx_ref[0, 0, :], (idx_ref.shape[2], 1))
    col = jax.lax.broadcasted_iota(jnp.int32, u.shape, 1)
    xu_ref[:, d_in:] = jnp.where((col >> 4) == idx, u, 0.0)
    o_ref[...] = jax.lax.dot_general(
        xu_ref[...], wb_ref[...], (((1,), (1,)), ((), ())),
        preferred_element_type=jnp.float32)


def kernel(x, active_lora_indices, W, lora_A_pool, lora_B_pool):
    n, d_in = x.shape
    d_out = W.shape[0]
    L, r, _ = lora_A_pool.shape
    lr = L * r
    nb = n // _N_BLK
    a_cat = lora_A_pool.reshape(lr, d_in)
    b_flat = (_SCALE * jnp.transpose(lora_B_pool, (1, 0, 2))).reshape(d_out, lr)
    idx3 = active_lora_indices.reshape(nb, 1, _N_BLK)
    return pl.pallas_call(
        functools.partial(_fused_body, d_in=d_in),
        grid=(nb,),
        in_specs=[
            pl.BlockSpec((1, 1, _N_BLK), lambda i: (i, 0, 0)),
            pl.BlockSpec((_N_BLK, d_in), lambda i: (i, 0)),
            pl.BlockSpec((d_out, d_in), lambda i: (0, 0)),
            pl.BlockSpec((lr, d_in), lambda i: (0, 0)),
            pl.BlockSpec((d_out, lr), lambda i: (0, 0)),
        ],
        out_specs=pl.BlockSpec((_N_BLK, d_out), lambda i: (i, 0)),
        out_shape=jax.ShapeDtypeStruct((n, d_out), jnp.float32),
        scratch_shapes=[pltpu.VMEM((d_out, d_in + lr), jnp.float32),
                        pltpu.VMEM((_N_BLK, d_in + lr), jnp.float32)],
        compiler_params=pltpu.CompilerParams(
            dimension_semantics=("arbitrary",)),
    )(idx3, x, W, a_cat, b_flat)
